# baseline (speedup 1.0000x reference)
import numpy as np

import jax
import jax.numpy as jnp
from jax import lax
from jax.experimental import pallas as pl
from jax.experimental.pallas import tpu as pltpu

_H0, _W0 = 24, 72
_H1, _W1 = 23, 71
_H2, _W2 = 22, 70
_C0 = 4
_S1S = 144
_P2W = 80
_D = 120
_DP = 128
_TB = 128
_SCALES = (1.0 / 2.0, 1.0 / 50.0, 1.0 / 200.0, 1.0 / 200.0)
_HW = _H0 * _W0


def _embed_kernel(x_ref, a1t_ref, a1m_ref, a1b_ref, b1_ref,
                  a2t_ref, a2m_ref, a2b_ref, b2_ref, wfc_ref, bfc_ref,
                  out_ref, xt_ref, s1_ref, p2_ref):
    f32 = jnp.float32
    bf16 = jnp.bfloat16

    for j in range(_C0 * _HW // _TB):
        xt_ref[j * _TB:(j + 1) * _TB, :] = (
            x_ref[:, j * _TB:(j + 1) * _TB].astype(bf16).T)

    dn = (((1,), (0,)), ((), ()))

    a1m = [a1m_ref[ci] for ci in range(_C0)]

    def conv1_row(y):
        if y == 0:
            parts = [lax.dot_general(a1t_ref[ci],
                                     xt_ref[ci * _HW: ci * _HW + 2 * _W0, :],
                                     dn, preferred_element_type=f32)
                     for ci in range(_C0)]
        elif y == _H0 - 1:
            parts = [lax.dot_general(a1b_ref[ci],
                                     xt_ref[ci * _HW + (_H0 - 2) * _W0:
                                            ci * _HW + _H0 * _W0, :],
                                     dn, preferred_element_type=f32)
                     for ci in range(_C0)]
        else:
            parts = [lax.dot_general(a1m[ci],
                                     xt_ref[ci * _HW + (y - 1) * _W0:
                                            ci * _HW + (y + 2) * _W0, :],
                                     dn, preferred_element_type=f32)
                     for ci in range(_C0)]
        return (parts[0] + parts[1]) + (parts[2] + parts[3])

    z1 = jnp.zeros((1, _TB), f32)
    z10 = jnp.zeros((_P2W - _W2, _TB), f32)
    b1_0 = b1_ref[0]
    b1_1 = b1_ref[1]

    prev = conv1_row(0)
    for h1 in range(_H1):
        cur = conv1_row(h1 + 1)
        vm = jnp.maximum(prev, cur)
        hp0 = jnp.maximum(vm[0:_W1, :], vm[1:_W1 + 1, :]) + b1_0
        hp1 = jnp.maximum(vm[_W0:_W0 + _W1, :],
                          vm[_W0 + 1:_W0 + _W1 + 1, :]) + b1_1
        s1_ref[h1 * _S1S:(h1 + 1) * _S1S, :] = jnp.concatenate(
            [hp0, z1, hp1, z1], axis=0).astype(bf16)
        prev = cur

    a2m = a2m_ref[...]

    def conv2_row(r):
        if r == 0:
            return lax.dot_general(a2t_ref[...], s1_ref[0:2 * _S1S, :],
                                   dn, preferred_element_type=f32)
        if r == _H2:
            return lax.dot_general(a2b_ref[...],
                                   s1_ref[(_H2 - 1) * _S1S:(_H2 + 1) * _S1S, :],
                                   dn, preferred_element_type=f32)
        return lax.dot_general(a2m, s1_ref[(r - 1) * _S1S:(r + 2) * _S1S, :],
                               dn, preferred_element_type=f32)

    b2_0 = b2_ref[0]
    prev2 = conv2_row(0)
    for h in range(_H2):
        cur2 = conv2_row(h + 1)
        vm = jnp.maximum(prev2, cur2)
        hp = jnp.maximum(vm[0:_W2, :], vm[1:_W2 + 1, :]) + b2_0
        p2_ref[h * _P2W:(h + 1) * _P2W, :] = jnp.concatenate(
            [hp, z10], axis=0).astype(bf16)
        prev2 = cur2

    out = lax.dot_general(p2_ref[...], wfc_ref[...],
                          (((0,), (0,)), ((), ())),
                          preferred_element_type=f32)
    out_ref[...] = out + bfc_ref[...]


def _band_matrices(w1, w2):
    f32 = jnp.float32
    scales = jnp.asarray(_SCALES, f32)
    w1s = w1.astype(f32) * scales[None, :, None, None]

    d1 = jnp.asarray(np.stack([np.eye(_W0, k=-1), np.eye(_W0), np.eye(_W0, k=1)]),
                     f32)
    band1 = jnp.einsum('ocyx,xab->ocyab', w1s, d1)
    a1m = band1.transpose(1, 0, 3, 2, 4).reshape(_C0, 2 * _W0, 3 * _W0)
    a1t = band1[:, :, 1:3].transpose(1, 0, 3, 2, 4).reshape(_C0, 2 * _W0, 2 * _W0)
    a1b = band1[:, :, 0:2].transpose(1, 0, 3, 2, 4).reshape(_C0, 2 * _W0, 2 * _W0)

    d2 = np.stack([np.eye(72, k=-1), np.eye(72), np.eye(72, k=1)])
    d2[:, _W1:, :] = 0.0
    d2[:, :, _W1:] = 0.0
    d2 = jnp.asarray(d2, f32)
    band2 = jnp.einsum('cyx,xab->cyab', w2[0].astype(f32), d2)
    a2m = band2.transpose(2, 1, 0, 3).reshape(72, 3 * _S1S)
    a2t = band2[:, 1:3].transpose(2, 1, 0, 3).reshape(72, 2 * _S1S)
    a2b = band2[:, 0:2].transpose(2, 1, 0, 3).reshape(72, 2 * _S1S)
    bf16 = jnp.bfloat16
    return (a1t.astype(bf16), a1m.astype(bf16), a1b.astype(bf16),
            a2t.astype(bf16), a2m.astype(bf16), a2b.astype(bf16))


def kernel(x, w1, b1, w2, b2, wfc, bfc):
    B, J, F = x.shape
    assert F == _C0 * _H0 * _W0
    N = B * J
    nb = pl.cdiv(N, _TB)
    Np = nb * _TB

    a1t, a1m, a1b, a2t, a2m, a2b = _band_matrices(w1, w2)

    wfc3 = jnp.zeros((_H2, _P2W, _DP), jnp.float32)
    wfc3 = wfc3.at[:, :_W2, :_D].set(
        wfc.T.astype(jnp.float32).reshape(_H2, _W2, _D))
    wfc_b = wfc3.reshape(_H2 * _P2W, _DP).astype(jnp.bfloat16)
    bfc_p = jnp.zeros((1, _DP), jnp.float32).at[0, :_D].set(bfc.astype(jnp.float32))

    xf = x.reshape(N, F).astype(jnp.float32)
    if Np != N:
        xf = jnp.pad(xf, ((0, Np - N), (0, 0)))

    out_p = pl.pallas_call(
        _embed_kernel,
        out_shape=jax.ShapeDtypeStruct((Np, _DP), jnp.float32),
        grid=(nb,),
        in_specs=[
            pl.BlockSpec((_TB, F), lambda i: (i, 0)),
            pl.BlockSpec((_C0, 2 * _W0, 2 * _W0), lambda i: (0, 0, 0)),
            pl.BlockSpec((_C0, 2 * _W0, 3 * _W0), lambda i: (0, 0, 0)),
            pl.BlockSpec((_C0, 2 * _W0, 2 * _W0), lambda i: (0, 0, 0)),
            pl.BlockSpec(memory_space=pltpu.MemorySpace.SMEM),
            pl.BlockSpec((72, 2 * _S1S), lambda i: (0, 0)),
            pl.BlockSpec((72, 3 * _S1S), lambda i: (0, 0)),
            pl.BlockSpec((72, 2 * _S1S), lambda i: (0, 0)),
            pl.BlockSpec(memory_space=pltpu.MemorySpace.SMEM),
            pl.BlockSpec((_H2 * _P2W, _DP), lambda i: (0, 0)),
            pl.BlockSpec((1, _DP), lambda i: (0, 0)),
        ],
        out_specs=pl.BlockSpec((_TB, _DP), lambda i: (i, 0)),
        scratch_shapes=[
            pltpu.VMEM((_C0 * _HW, _TB), jnp.bfloat16),
            pltpu.VMEM((_H1 * _S1S, _TB), jnp.bfloat16),
            pltpu.VMEM((_H2 * _P2W, _TB), jnp.bfloat16),
        ],
        compiler_params=pltpu.CompilerParams(
            dimension_semantics=("parallel",),
            vmem_limit_bytes=48 * 1024 * 1024,
        ),
    )(xf, a1t, a1m, a1b, b1.astype(jnp.float32),
      a2t, a2m, a2b, b2.astype(jnp.float32), wfc_b, bfc_p)

    return out_p[:N, :_D].reshape(B, J, _D)


# speedup vs baseline: 1.4776x; 1.4776x over previous
import numpy as np

import jax
import jax.numpy as jnp
from jax import lax
from jax.experimental import pallas as pl
from jax.experimental.pallas import tpu as pltpu

_H0, _W0 = 24, 72
_H1, _W1 = 23, 71
_H2, _W2 = 22, 70
_C0 = 4
_S1S = 144
_P2W = 80
_D = 120
_DP = 128
_TB = 256
_SCALES = (1.0 / 2.0, 1.0 / 50.0, 1.0 / 200.0, 1.0 / 200.0)
_HW = _H0 * _W0


def _embed_kernel(x_ref, a1t_ref, a1m_ref, a1b_ref, b1_ref,
                  a2t_ref, a2m_ref, a2b_ref, b2_ref, wfc_ref, bfc_ref,
                  out_ref, xt_ref, s1_ref, p2_ref):
    f32 = jnp.float32
    bf16 = jnp.bfloat16

    for j in range(_C0 * _HW // _TB):
        xt_ref[j * _TB:(j + 1) * _TB, :] = (
            x_ref[:, j * _TB:(j + 1) * _TB].astype(bf16).T)

    dn = (((1,), (0,)), ((), ()))

    a1m = [a1m_ref[ci] for ci in range(_C0)]

    def conv1_row(y):
        if y == 0:
            parts = [lax.dot_general(a1t_ref[ci],
                                     xt_ref[ci * _HW: ci * _HW + 2 * _W0, :],
                                     dn, preferred_element_type=f32)
                     for ci in range(_C0)]
        elif y == _H0 - 1:
            parts = [lax.dot_general(a1b_ref[ci],
                                     xt_ref[ci * _HW + (_H0 - 2) * _W0:
                                            ci * _HW + _H0 * _W0, :],
                                     dn, preferred_element_type=f32)
                     for ci in range(_C0)]
        else:
            parts = [lax.dot_general(a1m[ci],
                                     xt_ref[ci * _HW + (y - 1) * _W0:
                                            ci * _HW + (y + 2) * _W0, :],
                                     dn, preferred_element_type=f32)
                     for ci in range(_C0)]
        return (parts[0] + parts[1]) + (parts[2] + parts[3])

    z1 = jnp.zeros((1, _TB), f32)
    z10 = jnp.zeros((_P2W - _W2, _TB), f32)
    b1_0 = b1_ref[0]
    b1_1 = b1_ref[1]

    prev = conv1_row(0)
    for h1 in range(_H1):
        cur = conv1_row(h1 + 1)
        vm = jnp.maximum(prev, cur)
        hp0 = jnp.maximum(vm[0:_W1, :], vm[1:_W1 + 1, :]) + b1_0
        hp1 = jnp.maximum(vm[_W0:_W0 + _W1, :],
                          vm[_W0 + 1:_W0 + _W1 + 1, :]) + b1_1
        s1_ref[h1 * _S1S:(h1 + 1) * _S1S, :] = jnp.concatenate(
            [hp0, z1, hp1, z1], axis=0).astype(bf16)
        prev = cur

    a2m = a2m_ref[...]

    def conv2_row(r):
        if r == 0:
            return lax.dot_general(a2t_ref[...], s1_ref[0:2 * _S1S, :],
                                   dn, preferred_element_type=f32)
        if r == _H2:
            return lax.dot_general(a2b_ref[...],
                                   s1_ref[(_H2 - 1) * _S1S:(_H2 + 1) * _S1S, :],
                                   dn, preferred_element_type=f32)
        return lax.dot_general(a2m, s1_ref[(r - 1) * _S1S:(r + 2) * _S1S, :],
                               dn, preferred_element_type=f32)

    b2_0 = b2_ref[0]
    prev2 = conv2_row(0)
    for h in range(_H2):
        cur2 = conv2_row(h + 1)
        vm = jnp.maximum(prev2, cur2)
        hp = jnp.maximum(vm[0:_W2, :], vm[1:_W2 + 1, :]) + b2_0
        p2_ref[h * _P2W:(h + 1) * _P2W, :] = jnp.concatenate(
            [hp, z10], axis=0).astype(bf16)
        prev2 = cur2

    out = lax.dot_general(p2_ref[...], wfc_ref[...],
                          (((0,), (0,)), ((), ())),
                          preferred_element_type=f32)
    out_ref[...] = out + bfc_ref[...]


def _band_matrices(w1, w2):
    f32 = jnp.float32
    scales = jnp.asarray(_SCALES, f32)
    w1s = w1.astype(f32) * scales[None, :, None, None]

    d1 = jnp.asarray(np.stack([np.eye(_W0, k=-1), np.eye(_W0), np.eye(_W0, k=1)]),
                     f32)
    band1 = jnp.einsum('ocyx,xab->ocyab', w1s, d1)
    a1m = band1.transpose(1, 0, 3, 2, 4).reshape(_C0, 2 * _W0, 3 * _W0)
    a1t = band1[:, :, 1:3].transpose(1, 0, 3, 2, 4).reshape(_C0, 2 * _W0, 2 * _W0)
    a1b = band1[:, :, 0:2].transpose(1, 0, 3, 2, 4).reshape(_C0, 2 * _W0, 2 * _W0)

    d2 = np.stack([np.eye(72, k=-1), np.eye(72), np.eye(72, k=1)])
    d2[:, _W1:, :] = 0.0
    d2[:, :, _W1:] = 0.0
    d2 = jnp.asarray(d2, f32)
    band2 = jnp.einsum('cyx,xab->cyab', w2[0].astype(f32), d2)
    a2m = band2.transpose(2, 1, 0, 3).reshape(72, 3 * _S1S)
    a2t = band2[:, 1:3].transpose(2, 1, 0, 3).reshape(72, 2 * _S1S)
    a2b = band2[:, 0:2].transpose(2, 1, 0, 3).reshape(72, 2 * _S1S)
    bf16 = jnp.bfloat16
    return (a1t.astype(bf16), a1m.astype(bf16), a1b.astype(bf16),
            a2t.astype(bf16), a2m.astype(bf16), a2b.astype(bf16))


def kernel(x, w1, b1, w2, b2, wfc, bfc):
    B, J, F = x.shape
    assert F == _C0 * _H0 * _W0
    N = B * J
    nb = pl.cdiv(N, _TB)
    Np = nb * _TB

    a1t, a1m, a1b, a2t, a2m, a2b = _band_matrices(w1, w2)

    wfc3 = jnp.zeros((_H2, _P2W, _DP), jnp.float32)
    wfc3 = wfc3.at[:, :_W2, :_D].set(
        wfc.T.astype(jnp.float32).reshape(_H2, _W2, _D))
    wfc_b = wfc3.reshape(_H2 * _P2W, _DP).astype(jnp.bfloat16)
    bfc_p = jnp.zeros((1, _DP), jnp.float32).at[0, :_D].set(bfc.astype(jnp.float32))

    xf = x.reshape(N, F).astype(jnp.float32)
    if Np != N:
        xf = jnp.pad(xf, ((0, Np - N), (0, 0)))

    out_p = pl.pallas_call(
        _embed_kernel,
        out_shape=jax.ShapeDtypeStruct((Np, _DP), jnp.float32),
        grid=(nb,),
        in_specs=[
            pl.BlockSpec((_TB, F), lambda i: (i, 0)),
            pl.BlockSpec((_C0, 2 * _W0, 2 * _W0), lambda i: (0, 0, 0)),
            pl.BlockSpec((_C0, 2 * _W0, 3 * _W0), lambda i: (0, 0, 0)),
            pl.BlockSpec((_C0, 2 * _W0, 2 * _W0), lambda i: (0, 0, 0)),
            pl.BlockSpec(memory_space=pltpu.MemorySpace.SMEM),
            pl.BlockSpec((72, 2 * _S1S), lambda i: (0, 0)),
            pl.BlockSpec((72, 3 * _S1S), lambda i: (0, 0)),
            pl.BlockSpec((72, 2 * _S1S), lambda i: (0, 0)),
            pl.BlockSpec(memory_space=pltpu.MemorySpace.SMEM),
            pl.BlockSpec((_H2 * _P2W, _DP), lambda i: (0, 0)),
            pl.BlockSpec((1, _DP), lambda i: (0, 0)),
        ],
        out_specs=pl.BlockSpec((_TB, _DP), lambda i: (i, 0)),
        scratch_shapes=[
            pltpu.VMEM((_C0 * _HW, _TB), jnp.bfloat16),
            pltpu.VMEM((_H1 * _S1S, _TB), jnp.bfloat16),
            pltpu.VMEM((_H2 * _P2W, _TB), jnp.bfloat16),
        ],
        compiler_params=pltpu.CompilerParams(
            dimension_semantics=("parallel",),
            vmem_limit_bytes=48 * 1024 * 1024,
        ),
    )(xf, a1t, a1m, a1b, b1.astype(jnp.float32),
      a2t, a2m, a2b, b2.astype(jnp.float32), wfc_b, bfc_p)

    return out_p[:N, :_D].reshape(B, J, _D)
